# baseline (speedup 1.0000x reference)
import functools

import jax
import jax.numpy as jnp
from jax import lax
from jax.experimental import pallas as pl
from jax.experimental.pallas import tpu as pltpu
from jax.experimental.pallas import tpu_sc as plsc

B = 16384
IN_C = 67
OUT_C = 73
NW = 32
CPW = B // NW
L = 16


HALF = CPW // 2


def _sc_body(elev_t_hbm, tab_hbm, out_t_hbm,
             in_a, in_b, out_a, out_b, tab_v, sem_in, sem_out):
    wid = lax.axis_index("s") * 2 + lax.axis_index("c")
    col0 = pl.ds(wid * CPW, HALF)
    col1 = pl.ds(wid * CPW + HALF, HALF)

    cp_in0 = pltpu.make_async_copy(elev_t_hbm.at[:, col0], in_a, sem_in)
    cp_in1 = pltpu.make_async_copy(elev_t_hbm.at[:, col1], in_b, sem_in)
    cp_in0.start()
    cp_in1.start()
    pltpu.sync_copy(tab_hbm, tab_v)

    def shuffle(in_v, out_v):
        @plsc.parallel_loop(0, HALF // L, unroll=2)
        def chunk(j):
            sl = pl.ds(j * L, L)
            idx8 = in_v[18, sl].astype(jnp.int32) * 8
            for c in range(16):
                out_v[c, sl] = in_v[c, sl]
            out_v[16, sl] = in_v[17, sl]
            for c in range(17, 65):
                out_v[c, sl] = in_v[c + 2, sl]
            for e in range(8):
                out_v[65 + e, sl] = plsc.load_gather(tab_v, [idx8 + e])

    cp_in0.wait()
    shuffle(in_a, out_a)
    cp_out0 = pltpu.make_async_copy(out_a, out_t_hbm.at[:, col0], sem_out)
    cp_out0.start()
    cp_in1.wait()
    shuffle(in_b, out_b)
    cp_out1 = pltpu.make_async_copy(out_b, out_t_hbm.at[:, col1], sem_out)
    cp_out1.start()
    cp_out0.wait()
    cp_out1.wait()


_sc_kernel = functools.partial(
    pl.kernel,
    out_type=jax.ShapeDtypeStruct((OUT_C, B), jnp.float32),
    mesh=plsc.VectorSubcoreMesh(core_axis_name="c", subcore_axis_name="s"),
    compiler_params=pltpu.CompilerParams(
        needs_layout_passes=False, use_tc_tiling_on_sc=True,
        skip_device_barrier=True),
    scratch_types=[
        pltpu.VMEM((IN_C, HALF), jnp.float32),
        pltpu.VMEM((IN_C, HALF), jnp.float32),
        pltpu.VMEM((OUT_C, HALF), jnp.float32),
        pltpu.VMEM((OUT_C, HALF), jnp.float32),
        pltpu.VMEM((32,), jnp.float32),
        pltpu.SemaphoreType.DMA,
        pltpu.SemaphoreType.DMA,
    ],
)(_sc_body)


@jax.jit
def kernel(elev_info, door_table, srv_dir_table):
    del srv_dir_table
    out_t = _sc_kernel(elev_info.T, door_table.reshape(-1))
    return out_t.T


# speedup vs baseline: 1.0375x; 1.0375x over previous
import functools

import jax
import jax.numpy as jnp
from jax import lax
from jax.experimental import pallas as pl
from jax.experimental.pallas import tpu as pltpu
from jax.experimental.pallas import tpu_sc as plsc

B = 16384
IN_C = 67
OUT_C = 73
NW = 32
CPW = B // NW
L = 16


def _sc_body(elev_t_hbm, tab_hbm, out_t_hbm, in_v, out_v, tab_v):
    wid = lax.axis_index("s") * 2 + lax.axis_index("c")
    cols = pl.ds(wid * CPW, CPW)

    pltpu.sync_copy(elev_t_hbm.at[:, cols], in_v)
    pltpu.sync_copy(tab_hbm, tab_v)

    @plsc.parallel_loop(0, CPW // L)
    def chunk(j):
        sl = pl.ds(j * L, L)

        @plsc.parallel_loop(0, 16, unroll=4)
        def pos(c):
            out_v[c, sl] = in_v[c, sl]

        out_v[16, sl] = in_v[17, sl]

        @plsc.parallel_loop(17, 65, unroll=4)
        def calls(c):
            out_v[c, sl] = in_v[c + 2, sl]

        idx8 = in_v[18, sl].astype(jnp.int32) * 8
        for e in range(8):
            out_v[65 + e, sl] = plsc.load_gather(tab_v, [idx8 + e])

    pltpu.sync_copy(out_v, out_t_hbm.at[:, cols])


_sc_kernel = functools.partial(
    pl.kernel,
    out_type=jax.ShapeDtypeStruct((OUT_C, B), jnp.float32),
    mesh=plsc.VectorSubcoreMesh(core_axis_name="c", subcore_axis_name="s"),
    compiler_params=pltpu.CompilerParams(
        needs_layout_passes=False, use_tc_tiling_on_sc=True,
        skip_device_barrier=True),
    scratch_types=[
        pltpu.VMEM((IN_C, CPW), jnp.float32),
        pltpu.VMEM((OUT_C, CPW), jnp.float32),
        pltpu.VMEM((32,), jnp.float32),
    ],
)(_sc_body)


@jax.jit
def kernel(elev_info, door_table, srv_dir_table):
    del srv_dir_table
    out_t = _sc_kernel(elev_info.T, door_table.reshape(-1))
    return out_t.T


# speedup vs baseline: 1.0638x; 1.0253x over previous
import functools

import jax
import jax.numpy as jnp
from jax import lax
from jax.experimental import pallas as pl
from jax.experimental.pallas import tpu as pltpu
from jax.experimental.pallas import tpu_sc as plsc

B = 16384
IN_C = 67
OUT_C = 73
NW = 32
CPW = B // NW
L = 16


def _sc_body(elev_t_hbm, tab_hbm, out_t_hbm, in_v, out_v, tab_v):
    wid = lax.axis_index("s") * 2 + lax.axis_index("c")
    cols = pl.ds(wid * CPW, CPW)

    pltpu.sync_copy(elev_t_hbm.at[:, cols], in_v)
    pltpu.sync_copy(tab_hbm, tab_v)

    @plsc.parallel_loop(0, CPW // L, unroll=2)
    def chunk(j):
        sl = pl.ds(j * L, L)
        idx = in_v[18, sl].astype(jnp.int32)
        for c in range(16):
            out_v[c, sl] = in_v[c, sl]
        out_v[16, sl] = in_v[17, sl]
        for c in range(17, 65):
            out_v[c, sl] = in_v[c + 2, sl]
        for e in range(8):
            out_v[65 + e, sl] = plsc.load_gather(
                tab_v, [idx, jnp.full((L,), e, jnp.int32)])

    pltpu.sync_copy(out_v, out_t_hbm.at[:, cols])


_sc_kernel = functools.partial(
    pl.kernel,
    out_type=jax.ShapeDtypeStruct((OUT_C, B), jnp.float32),
    mesh=plsc.VectorSubcoreMesh(core_axis_name="c", subcore_axis_name="s"),
    compiler_params=pltpu.CompilerParams(
        needs_layout_passes=False, use_tc_tiling_on_sc=True,
        skip_device_barrier=True),
    scratch_types=[
        pltpu.VMEM((IN_C, CPW), jnp.float32),
        pltpu.VMEM((OUT_C, CPW), jnp.float32),
        pltpu.VMEM((4, 8), jnp.float32),
    ],
)(_sc_body)


@jax.jit
def kernel(elev_info, door_table, srv_dir_table):
    del srv_dir_table
    out_t = _sc_kernel(elev_info.T, door_table)
    return out_t.T


# speedup vs baseline: 1.1137x; 1.0469x over previous
import functools

import jax
import jax.numpy as jnp
from jax import lax
from jax.experimental import pallas as pl
from jax.experimental.pallas import tpu as pltpu
from jax.experimental.pallas import tpu_sc as plsc

B = 16384
IN_C = 67
OUT_C = 73
NW = 32
CPW = B // NW
L = 16


NQ = 4
QW = CPW // NQ


def _sc_body(elev_t_hbm, tab_hbm, out_t_hbm, in_v, out_v, tab_v,
             sem_in, sem_out):
    wid = lax.axis_index("s") * 2 + lax.axis_index("c")
    base = wid * CPW

    def in_cp(q):
        return pltpu.make_async_copy(
            elev_t_hbm.at[:, pl.ds(base + q * QW, QW)],
            in_v.at[:, pl.ds(q * QW, QW)], sem_in)

    def out_cp(q):
        return pltpu.make_async_copy(
            out_v.at[:, pl.ds(q * QW, QW)],
            out_t_hbm.at[:, pl.ds(base + q * QW, QW)], sem_out)

    for q in range(NQ):
        in_cp(q).start()
    pltpu.sync_copy(tab_hbm, tab_v)

    def quarter(q, carry):
        in_cp(q).wait()

        @plsc.parallel_loop(0, QW // L, unroll=2)
        def chunk(j):
            sl = pl.ds(q * QW + j * L, L)
            idx = in_v[18, sl].astype(jnp.int32)
            for c in range(16):
                out_v[c, sl] = in_v[c, sl]
            out_v[16, sl] = in_v[17, sl]
            for c in range(17, 65):
                out_v[c, sl] = in_v[c + 2, sl]
            for e in range(8):
                out_v[65 + e, sl] = plsc.load_gather(
                    tab_v, [idx, jnp.full((L,), e, jnp.int32)])

        out_cp(q).start()
        return carry

    lax.fori_loop(0, NQ, quarter, 0)
    for q in range(NQ):
        out_cp(q).wait()


_sc_kernel = functools.partial(
    pl.kernel,
    out_type=jax.ShapeDtypeStruct((OUT_C, B), jnp.float32),
    mesh=plsc.VectorSubcoreMesh(core_axis_name="c", subcore_axis_name="s"),
    compiler_params=pltpu.CompilerParams(
        needs_layout_passes=False, use_tc_tiling_on_sc=True,
        skip_device_barrier=True),
    scratch_types=[
        pltpu.VMEM((IN_C, CPW), jnp.float32),
        pltpu.VMEM((OUT_C, CPW), jnp.float32),
        pltpu.VMEM((4, 8), jnp.float32),
        pltpu.SemaphoreType.DMA,
        pltpu.SemaphoreType.DMA,
    ],
)(_sc_body)


@jax.jit
def kernel(elev_info, door_table, srv_dir_table):
    del srv_dir_table
    out_t = _sc_kernel(elev_info.T, door_table)
    return out_t.T


# speedup vs baseline: 1.1210x; 1.0065x over previous
import functools

import jax
import jax.numpy as jnp
from jax import lax
from jax.experimental import pallas as pl
from jax.experimental.pallas import tpu as pltpu
from jax.experimental.pallas import tpu_sc as plsc

B = 16384
IN_C = 67
OUT_C = 73
NW = 32
CPW = B // NW
L = 16


NQ = 4
QW = CPW // NQ


def _sc_body(elev_t_hbm, tab_hbm, out_t_hbm, in_v, out_v, tab_v,
             sem_in, sem_out):
    wid = lax.axis_index("s") * 2 + lax.axis_index("c")
    base = wid * CPW

    def in_cp(q):
        return pltpu.make_async_copy(
            elev_t_hbm.at[:, pl.ds(base + q * QW, QW)],
            in_v.at[:, pl.ds(q * QW, QW)], sem_in)

    def out_cp(q):
        return pltpu.make_async_copy(
            out_v.at[:, pl.ds(q * QW, QW)],
            out_t_hbm.at[:, pl.ds(base + q * QW, QW)], sem_out)

    for q in range(NQ):
        in_cp(q).start()
    pltpu.sync_copy(tab_hbm, tab_v)

    def quarter(q, carry):
        in_cp(q).wait()

        @plsc.parallel_loop(0, QW // L, unroll=4)
        def chunk(j):
            sl = pl.ds(q * QW + j * L, L)
            idx = in_v[18, sl].astype(jnp.int32)
            for c in range(16):
                out_v[c, sl] = in_v[c, sl]
            out_v[16, sl] = in_v[17, sl]
            for c in range(17, 65):
                out_v[c, sl] = in_v[c + 2, sl]
            for e in range(8):
                out_v[65 + e, sl] = plsc.load_gather(
                    tab_v, [idx, jnp.full((L,), e, jnp.int32)])

        out_cp(q).start()
        return carry

    lax.fori_loop(0, NQ, quarter, 0)
    for q in range(NQ):
        out_cp(q).wait()


_sc_kernel = functools.partial(
    pl.kernel,
    out_type=jax.ShapeDtypeStruct((OUT_C, B), jnp.float32),
    mesh=plsc.VectorSubcoreMesh(core_axis_name="c", subcore_axis_name="s"),
    compiler_params=pltpu.CompilerParams(
        needs_layout_passes=False, use_tc_tiling_on_sc=True,
        skip_device_barrier=True),
    scratch_types=[
        pltpu.VMEM((IN_C, CPW), jnp.float32),
        pltpu.VMEM((OUT_C, CPW), jnp.float32),
        pltpu.VMEM((4, 8), jnp.float32),
        pltpu.SemaphoreType.DMA,
        pltpu.SemaphoreType.DMA,
    ],
)(_sc_body)


@jax.jit
def kernel(elev_info, door_table, srv_dir_table):
    del srv_dir_table
    out_t = _sc_kernel(elev_info.T, door_table)
    return out_t.T
